# Initial kernel scaffold; baseline (speedup 1.0000x reference)
import functools

import jax
import jax.numpy as jnp
from jax import lax
from jax.experimental import pallas as pl
from jax.experimental.pallas import tpu as pltpu
from jax.experimental.pallas import tpu_sc as plsc

LEN_RULES = 100000
NUM_REL_RULES = 50000
PAD_TOK = NUM_REL_RULES
BATCH = 16384
HIST = 200

NC, NS, L = 2, 16, 16
NW = NC * NS

G_PAD = 100352
G_CHUNK = G_PAD // NW
ROWS_W = BATCH // NW
GROUPS = ROWS_W // L
BLK = L * HIST

_mesh = plsc.VectorSubcoreMesh(core_axis_name="c", subcore_axis_name="s")


def _wid():
    return lax.axis_index("s") * NC + lax.axis_index("c")


@functools.partial(
    pl.kernel,
    out_type=jax.ShapeDtypeStruct((G_PAD,), jnp.float32),
    mesh=_mesh,
    scratch_types=[
        pltpu.VMEM((NUM_REL_RULES + 1,), jnp.float32),
        pltpu.VMEM((G_CHUNK,), jnp.int32),
        pltpu.VMEM((G_CHUNK,), jnp.float32),
    ],
)
def _fuse_tables(g2l_hbm, table_hbm, fused_hbm, tab_v, idx_v, out_v):
    w = _wid()
    pltpu.sync_copy(table_hbm, tab_v)
    pltpu.sync_copy(g2l_hbm.at[pl.ds(w * G_CHUNK, G_CHUNK)], idx_v)

    def body(i, carry):
        idx = idx_v[pl.ds(i * L, L)]
        out_v[pl.ds(i * L, L)] = plsc.load_gather(tab_v, [idx])
        return carry

    lax.fori_loop(0, G_CHUNK // L, body, 0)
    pltpu.sync_copy(out_v, fused_hbm.at[pl.ds(w * G_CHUNK, G_CHUNK)])


@functools.partial(
    pl.kernel,
    out_type=jax.ShapeDtypeStruct((BATCH,), jnp.float32),
    mesh=_mesh,
    scratch_types=[
        pltpu.VMEM((G_PAD,), jnp.float32),
        pltpu.VMEM((2, BLK), jnp.int32),
        pltpu.VMEM((ROWS_W,), jnp.float32),
        pltpu.VMEM((1, 1), jnp.float32),
        pltpu.SemaphoreType.DMA,
        pltpu.SemaphoreType.DMA,
    ],
)
def _aggregate(rules_hbm, fused_hbm, bias_hbm, out_hbm,
               tab_v, idx_v, out_v, bias_v, sem0, sem1):
    w = _wid()
    base = w * ROWS_W * HIST
    pltpu.sync_copy(bias_hbm, bias_v)
    pltpu.sync_copy(fused_hbm, tab_v)
    bias = bias_v[0, 0]
    rowoffs = lax.iota(jnp.int32, L) * HIST

    cp0 = pltpu.async_copy(rules_hbm.at[pl.ds(base, BLK)], idx_v.at[0], sem0)
    cp1 = pltpu.async_copy(rules_hbm.at[pl.ds(base + BLK, BLK)], idx_v.at[1],
                           sem1)
    cp0.wait()
    cp1.wait()

    def group(g, buf, sem):
        bsplat = jnp.full((L,), buf, jnp.int32)

        def hist(l, acc):
            idx = plsc.load_gather(idx_v, [bsplat, rowoffs + l])
            return acc + plsc.load_gather(tab_v, [idx])

        acc = lax.fori_loop(0, HIST, hist, jnp.zeros((L,), jnp.float32))
        out_v[pl.ds(g * L, L)] = acc + bias

        @pl.when(g + 2 < GROUPS)
        def _():
            pltpu.async_copy(
                rules_hbm.at[pl.ds(base + (g + 2) * BLK, BLK)],
                idx_v.at[buf], sem).wait()

    def pair(i, carry):
        group(i * 2, 0, sem0)
        group(i * 2 + 1, 1, sem1)
        return carry

    lax.fori_loop(0, GROUPS // 2, pair, 0)
    pltpu.sync_copy(out_v, out_hbm.at[pl.ds(w * ROWS_W, ROWS_W)])


def kernel(rules, global_to_local, rules_table, bias):
    g2l_pad = jnp.concatenate([
        global_to_local,
        jnp.full((G_PAD - LEN_RULES - 1,), PAD_TOK, jnp.int32),
    ])
    fused = _fuse_tables(g2l_pad, rules_table.reshape(-1))
    out = _aggregate(rules.reshape(-1), fused, bias)
    return out.reshape(BATCH, 1)


# speedup vs baseline: 519.4118x; 519.4118x over previous
import functools

import jax
import jax.numpy as jnp
from jax import lax
from jax.experimental import pallas as pl
from jax.experimental.pallas import tpu as pltpu
from jax.experimental.pallas import tpu_sc as plsc

LEN_RULES = 100000
NUM_REL_RULES = 50000
PAD_TOK = NUM_REL_RULES
BATCH = 16384
HIST = 200

NC, NS, L = 2, 16, 16
NW = NC * NS

G_PAD = 100352
G_CHUNK = G_PAD // NW
ROWS_W = BATCH // NW
GROUPS = ROWS_W // L
BLK = L * HIST

_mesh = plsc.VectorSubcoreMesh(core_axis_name="c", subcore_axis_name="s")
_params = pltpu.CompilerParams(needs_layout_passes=False)


def _wid():
    return lax.axis_index("s") * NC + lax.axis_index("c")


@functools.partial(
    pl.kernel,
    out_type=jax.ShapeDtypeStruct((G_PAD,), jnp.float32),
    mesh=_mesh,
    scratch_types=[
        pltpu.VMEM((NUM_REL_RULES + 1,), jnp.float32),
        pltpu.VMEM((G_CHUNK,), jnp.int32),
        pltpu.VMEM((G_CHUNK,), jnp.float32),
    ],
    compiler_params=_params,
)
def _fuse_tables(g2l_hbm, table_hbm, fused_hbm, tab_v, idx_v, out_v):
    w = _wid()
    pltpu.sync_copy(table_hbm, tab_v)
    pltpu.sync_copy(g2l_hbm.at[pl.ds(w * G_CHUNK, G_CHUNK)], idx_v)

    def body(i, carry):
        idx = idx_v[pl.ds(i * L, L)]
        out_v[pl.ds(i * L, L)] = plsc.load_gather(tab_v, [idx])
        return carry

    lax.fori_loop(0, G_CHUNK // L, body, 0)
    pltpu.sync_copy(out_v, fused_hbm.at[pl.ds(w * G_CHUNK, G_CHUNK)])


@functools.partial(
    pl.kernel,
    out_type=jax.ShapeDtypeStruct((BATCH,), jnp.float32),
    mesh=_mesh,
    scratch_types=[
        pltpu.VMEM((G_PAD,), jnp.float32),
        pltpu.VMEM((2, BLK), jnp.int32),
        pltpu.VMEM((ROWS_W,), jnp.float32),
        pltpu.VMEM((L,), jnp.float32),
        pltpu.SemaphoreType.DMA,
        pltpu.SemaphoreType.DMA,
    ],
    compiler_params=_params,
)
def _aggregate(rules_hbm, fused_hbm, bias_hbm, out_hbm,
               tab_v, idx_v, out_v, bias_v, sem0, sem1):
    w = _wid()
    base = w * ROWS_W * HIST
    pltpu.sync_copy(bias_hbm, bias_v)
    pltpu.sync_copy(fused_hbm, tab_v)
    bias = bias_v[...]
    rowoffs = lax.iota(jnp.int32, L) * HIST

    def blk_src(g):
        return rules_hbm.at[pl.ds(base + g * BLK, BLK)]

    pltpu.async_copy(blk_src(0), idx_v.at[0], sem0)

    def group(g, buf, sem, next_sem):
        pltpu.make_async_copy(blk_src(g), idx_v.at[buf], sem).wait()

        @pl.when(g + 1 < GROUPS)
        def _():
            pltpu.async_copy(blk_src(g + 1), idx_v.at[1 - buf], next_sem)

        bsplat = jnp.full((L,), buf, jnp.int32)

        def hist(l, acc):
            idx = plsc.load_gather(idx_v, [bsplat, rowoffs + l])
            return acc + plsc.load_gather(tab_v, [idx])

        acc = lax.fori_loop(0, HIST, hist, jnp.zeros((L,), jnp.float32))
        out_v[pl.ds(g * L, L)] = acc + bias

    def pair(i, carry):
        group(i * 2, 0, sem0, sem1)
        group(i * 2 + 1, 1, sem1, sem0)
        return carry

    lax.fori_loop(0, GROUPS // 2, pair, 0)
    pltpu.sync_copy(out_v, out_hbm.at[pl.ds(w * ROWS_W, ROWS_W)])


def kernel(rules, global_to_local, rules_table, bias):
    g2l_pad = jnp.concatenate([
        global_to_local,
        jnp.full((G_PAD - LEN_RULES - 1,), PAD_TOK, jnp.int32),
    ])
    fused = _fuse_tables(g2l_pad, rules_table.reshape(-1))
    bias_vec = jnp.broadcast_to(bias.reshape(()), (L,))
    out = _aggregate(rules.reshape(-1), fused, bias_vec)
    return out.reshape(BATCH, 1)


# speedup vs baseline: 657.8752x; 1.2666x over previous
import functools

import jax
import jax.numpy as jnp
from jax import lax
from jax.experimental import pallas as pl
from jax.experimental.pallas import tpu as pltpu
from jax.experimental.pallas import tpu_sc as plsc

LEN_RULES = 100000
NUM_REL_RULES = 50000
PAD_TOK = NUM_REL_RULES
BATCH = 16384
HIST = 200

NC, NS, L = 2, 16, 16
NW = NC * NS

G_PAD = 100352
G_CHUNK = G_PAD // NW
ROWS_W = BATCH // NW
GROUPS = ROWS_W // L
BLK = L * HIST

_mesh = plsc.VectorSubcoreMesh(core_axis_name="c", subcore_axis_name="s")
_params = pltpu.CompilerParams(needs_layout_passes=False)


def _wid():
    return lax.axis_index("s") * NC + lax.axis_index("c")


@functools.partial(
    pl.kernel,
    out_type=jax.ShapeDtypeStruct((G_PAD,), jnp.float32),
    mesh=_mesh,
    scratch_types=[
        pltpu.VMEM((NUM_REL_RULES + 1,), jnp.float32),
        pltpu.VMEM((G_CHUNK,), jnp.int32),
        pltpu.VMEM((G_CHUNK,), jnp.float32),
    ],
    compiler_params=_params,
)
def _fuse_tables(g2l_hbm, table_hbm, fused_hbm, tab_v, idx_v, out_v):
    w = _wid()
    pltpu.sync_copy(table_hbm, tab_v)
    pltpu.sync_copy(g2l_hbm.at[pl.ds(w * G_CHUNK, G_CHUNK)], idx_v)

    def body(i, carry):
        idx = idx_v[pl.ds(i * L, L)]
        out_v[pl.ds(i * L, L)] = plsc.load_gather(tab_v, [idx])
        return carry

    lax.fori_loop(0, G_CHUNK // L, body, 0)
    pltpu.sync_copy(out_v, fused_hbm.at[pl.ds(w * G_CHUNK, G_CHUNK)])


@functools.partial(
    pl.kernel,
    out_type=jax.ShapeDtypeStruct((BATCH,), jnp.float32),
    mesh=_mesh,
    scratch_types=[
        pltpu.VMEM((G_PAD,), jnp.float32),
        pltpu.VMEM((2 * BLK,), jnp.int32),
        pltpu.VMEM((ROWS_W,), jnp.float32),
        pltpu.VMEM((L,), jnp.float32),
        pltpu.SemaphoreType.DMA,
        pltpu.SemaphoreType.DMA,
    ],
    compiler_params=_params,
)
def _aggregate(rules_hbm, fused_hbm, bias_hbm, out_hbm,
               tab_v, idx_v, out_v, bias_v, sem0, sem1):
    w = _wid()
    base = w * ROWS_W * HIST
    pltpu.sync_copy(bias_hbm, bias_v)
    pltpu.sync_copy(fused_hbm, tab_v)
    bias = bias_v[...]
    rowoffs = lax.iota(jnp.int32, L) * HIST

    def blk_src(g):
        return rules_hbm.at[pl.ds(base + g * BLK, BLK)]

    pltpu.async_copy(blk_src(0), idx_v.at[pl.ds(0, BLK)], sem0)

    UNROLL = 8

    def group(g, buf, sem, next_sem):
        pltpu.make_async_copy(blk_src(g), idx_v.at[pl.ds(buf * BLK, BLK)],
                              sem).wait()

        @pl.when(g + 1 < GROUPS)
        def _():
            pltpu.async_copy(blk_src(g + 1),
                             idx_v.at[pl.ds((1 - buf) * BLK, BLK)], next_sem)

        boffs = rowoffs + buf * BLK

        def hist(i, acc):
            l0 = i * UNROLL
            for u in range(UNROLL):
                idx = plsc.load_gather(idx_v, [boffs + (l0 + u)])
                acc = acc + plsc.load_gather(tab_v, [idx])
            return acc

        acc = lax.fori_loop(0, HIST // UNROLL, hist,
                            jnp.zeros((L,), jnp.float32))
        out_v[pl.ds(g * L, L)] = acc + bias

    def pair(i, carry):
        group(i * 2, 0, sem0, sem1)
        group(i * 2 + 1, 1, sem1, sem0)
        return carry

    lax.fori_loop(0, GROUPS // 2, pair, 0)
    pltpu.sync_copy(out_v, out_hbm.at[pl.ds(w * ROWS_W, ROWS_W)])


def kernel(rules, global_to_local, rules_table, bias):
    g2l_pad = jnp.concatenate([
        global_to_local,
        jnp.full((G_PAD - LEN_RULES - 1,), PAD_TOK, jnp.int32),
    ])
    fused = _fuse_tables(g2l_pad, rules_table.reshape(-1))
    bias_vec = jnp.broadcast_to(bias.reshape(()), (L,))
    out = _aggregate(rules.reshape(-1), fused, bias_vec)
    return out.reshape(BATCH, 1)


# speedup vs baseline: 732.4390x; 1.1133x over previous
import functools

import jax
import jax.numpy as jnp
from jax import lax
from jax.experimental import pallas as pl
from jax.experimental.pallas import tpu as pltpu
from jax.experimental.pallas import tpu_sc as plsc

LEN_RULES = 100000
NUM_REL_RULES = 50000
BATCH = 16384
HIST = 200

NC, NS, L = 2, 16, 16
NW = NC * NS

G_PAD = 100352
G_CHUNK = G_PAD // NW
LAST_OFF = (NW - 1) * G_CHUNK
LAST_N = LEN_RULES - LAST_OFF
ROWS_W = BATCH // NW
GROUPS = ROWS_W // L
BLK = L * HIST
NBUF = 4

_mesh = plsc.VectorSubcoreMesh(core_axis_name="c", subcore_axis_name="s")
_params = pltpu.CompilerParams(needs_layout_passes=False)


def _wid():
    return lax.axis_index("s") * NC + lax.axis_index("c")


@functools.partial(
    pl.kernel,
    out_type=jax.ShapeDtypeStruct((G_PAD,), jnp.float32),
    mesh=_mesh,
    scratch_types=[
        pltpu.VMEM((NUM_REL_RULES + 1,), jnp.float32),
        pltpu.VMEM((G_CHUNK,), jnp.int32),
        pltpu.VMEM((G_CHUNK,), jnp.float32),
    ],
    compiler_params=_params,
)
def _fuse_tables(g2l_hbm, table_hbm, fused_hbm, tab_v, idx_v, out_v):
    w = _wid()
    pltpu.sync_copy(table_hbm, tab_v)

    def chunk(off, n):
        pltpu.sync_copy(g2l_hbm.at[pl.ds(off, n)], idx_v.at[pl.ds(0, n)])

        def body(i, carry):
            idx = idx_v[pl.ds(i * L, L)]
            out_v[pl.ds(i * L, L)] = plsc.load_gather(tab_v, [idx])
            return carry

        lax.fori_loop(0, n // L, body, 0)
        pltpu.sync_copy(out_v.at[pl.ds(0, n)], fused_hbm.at[pl.ds(off, n)])

    @pl.when(w < NW - 1)
    def _():
        chunk(w * G_CHUNK, G_CHUNK)

    @pl.when(w == NW - 1)
    def _():
        chunk(LAST_OFF, LAST_N)


@functools.partial(
    pl.kernel,
    out_type=jax.ShapeDtypeStruct((BATCH,), jnp.float32),
    mesh=_mesh,
    scratch_types=[
        pltpu.VMEM((G_PAD,), jnp.float32),
        pltpu.VMEM((NBUF * BLK,), jnp.int32),
        pltpu.VMEM((ROWS_W,), jnp.float32),
        pltpu.VMEM((L,), jnp.float32),
        [pltpu.SemaphoreType.DMA] * NBUF,
    ],
    compiler_params=_params,
)
def _aggregate(rules_hbm, fused_hbm, bias_hbm, out_hbm,
               tab_v, idx_v, out_v, bias_v, sems):
    w = _wid()
    base = w * ROWS_W * HIST
    pltpu.sync_copy(bias_hbm, bias_v)
    pltpu.sync_copy(fused_hbm, tab_v)
    bias = bias_v[...]
    rowoffs = lax.iota(jnp.int32, L) * HIST

    def blk_src(g):
        return rules_hbm.at[pl.ds(base + g * BLK, BLK)]

    def blk_dst(b):
        return idx_v.at[pl.ds(b * BLK, BLK)]

    for b in range(NBUF - 1):
        pltpu.async_copy(blk_src(b), blk_dst(b), sems[b])

    UNROLL = 8

    def step(g, buf):
        pltpu.make_async_copy(blk_src(g), blk_dst(buf), sems[buf]).wait()
        nbuf = (buf + NBUF - 1) % NBUF

        @pl.when(g + NBUF - 1 < GROUPS)
        def _():
            pltpu.async_copy(blk_src(g + NBUF - 1), blk_dst(nbuf), sems[nbuf])

        boffs = rowoffs + buf * BLK

        def hist(i, acc):
            l0 = i * UNROLL
            for u in range(UNROLL):
                idx = plsc.load_gather(idx_v, [boffs + (l0 + u)])
                acc = acc + plsc.load_gather(tab_v, [idx])
            return acc

        acc = lax.fori_loop(0, HIST // UNROLL, hist,
                            jnp.zeros((L,), jnp.float32))
        out_v[pl.ds(g * L, L)] = acc + bias

    def ring(i, carry):
        for b in range(NBUF):
            step(i * NBUF + b, b)
        return carry

    lax.fori_loop(0, GROUPS // NBUF, ring, 0)
    pltpu.sync_copy(out_v, out_hbm.at[pl.ds(w * ROWS_W, ROWS_W)])


def kernel(rules, global_to_local, rules_table, bias):
    fused = _fuse_tables(global_to_local, rules_table.reshape(-1))
    bias_vec = jnp.broadcast_to(bias.reshape(()), (L,))
    out = _aggregate(rules.reshape(-1), fused, bias_vec)
    return out.reshape(BATCH, 1)


# speedup vs baseline: 749.9487x; 1.0239x over previous
import functools

import jax
import jax.numpy as jnp
from jax import lax
from jax.experimental import pallas as pl
from jax.experimental.pallas import tpu as pltpu
from jax.experimental.pallas import tpu_sc as plsc

LEN_RULES = 100000
NUM_REL_RULES = 50000
BATCH = 16384
HIST = 200

NC, NS, L = 2, 16, 16
NW = NC * NS

G_PAD = 100352
S_CHUNK = G_PAD // NS
LAST_OFF = (NS - 1) * S_CHUNK
LAST_N = LEN_RULES - LAST_OFF
ROWS_W = BATCH // NW
GROUPS = ROWS_W // L
BLK = L * HIST
NBUF = 3

_mesh = plsc.VectorSubcoreMesh(core_axis_name="c", subcore_axis_name="s")
_params = pltpu.CompilerParams(needs_layout_passes=False)


@functools.partial(
    pl.kernel,
    out_type=jax.ShapeDtypeStruct((BATCH,), jnp.float32),
    mesh=_mesh,
    scratch_types=[
        pltpu.VMEM((G_PAD,), jnp.float32),
        pltpu.VMEM((NBUF * BLK,), jnp.int32),
        pltpu.VMEM((S_CHUNK,), jnp.int32),
        pltpu.VMEM((S_CHUNK,), jnp.float32),
        pltpu.VMEM((ROWS_W,), jnp.float32),
        pltpu.VMEM((L,), jnp.float32),
        pltpu.VMEM_SHARED((G_PAD,), jnp.float32),
        [pltpu.SemaphoreType.DMA] * NBUF,
    ],
    compiler_params=_params,
)
def _run(rules_hbm, g2l_hbm, table_hbm, bias_hbm, out_hbm,
         tab_v, idx_v, g2l_v, fchunk_v, out_v, bias_v, fused_sh, sems):
    s = lax.axis_index("s")
    w = s * NC + lax.axis_index("c")
    base = w * ROWS_W * HIST

    def blk_src(g):
        return rules_hbm.at[pl.ds(base + g * BLK, BLK)]

    def blk_dst(b):
        return idx_v.at[pl.ds(b * BLK, BLK)]

    for b in range(NBUF - 1):
        pltpu.async_copy(blk_src(b), blk_dst(b), sems[b])

    pltpu.sync_copy(bias_hbm, bias_v)
    pltpu.sync_copy(table_hbm, tab_v.at[pl.ds(0, NUM_REL_RULES + 1)])

    def fuse_chunk(off, n):
        pltpu.sync_copy(g2l_hbm.at[pl.ds(off, n)], g2l_v.at[pl.ds(0, n)])

        def body(i, carry):
            idx = g2l_v[pl.ds(i * L, L)]
            fchunk_v[pl.ds(i * L, L)] = plsc.load_gather(tab_v, [idx])
            return carry

        lax.fori_loop(0, n // L, body, 0)
        pltpu.sync_copy(fchunk_v.at[pl.ds(0, n)], fused_sh.at[pl.ds(off, n)])

    @pl.when(s < NS - 1)
    def _():
        fuse_chunk(s * S_CHUNK, S_CHUNK)

    @pl.when(s == NS - 1)
    def _():
        fuse_chunk(LAST_OFF, LAST_N)

    plsc.subcore_barrier()
    pltpu.sync_copy(fused_sh, tab_v)

    bias = bias_v[...]
    rowoffs = lax.iota(jnp.int32, L) * HIST
    UNROLL = 8

    def step(g, buf, queue_ahead):
        pltpu.make_async_copy(blk_src(g), blk_dst(buf), sems[buf]).wait()
        nbuf = (buf + NBUF - 1) % NBUF

        if queue_ahead:
            @pl.when(g + NBUF - 1 < GROUPS)
            def _():
                pltpu.async_copy(blk_src(g + NBUF - 1), blk_dst(nbuf),
                                 sems[nbuf])

        boffs = rowoffs + buf * BLK

        def hist(i, acc):
            l0 = i * UNROLL
            for u in range(UNROLL):
                idx = plsc.load_gather(idx_v, [boffs + (l0 + u)])
                acc = acc + plsc.load_gather(tab_v, [idx])
            return acc

        acc = lax.fori_loop(0, HIST // UNROLL, hist,
                            jnp.zeros((L,), jnp.float32))
        out_v[pl.ds(g * L, L)] = acc + bias

    def ring(i, carry):
        for b in range(NBUF):
            step(i * NBUF + b, b, True)
        return carry

    FULL = (GROUPS // NBUF) * NBUF
    lax.fori_loop(0, GROUPS // NBUF, ring, 0)
    for g in range(FULL, GROUPS):
        step(g, g % NBUF, False)
    pltpu.sync_copy(out_v, out_hbm.at[pl.ds(w * ROWS_W, ROWS_W)])


def kernel(rules, global_to_local, rules_table, bias):
    bias_vec = jnp.broadcast_to(bias.reshape(()), (L,))
    out = _run(rules.reshape(-1), global_to_local, rules_table.reshape(-1),
               bias_vec)
    return out.reshape(BATCH, 1)
